# baseline (speedup 1.0000x reference)
import functools

import jax
import jax.numpy as jnp
from jax import lax
from jax.experimental import pallas as pl
from jax.experimental.pallas import tpu as pltpu
from jax.experimental.pallas import tpu_sc as plsc

N = 8192
K = 9
D_TBL = 256
KNN_R = 256
MLP_NB = 512

_NC, _NS = 2, 16
_NW = _NC * _NS
_GCH = 128


def _leaky(x):
    return jnp.where(x > 0, x, 0.1 * x)


def _knn_body(xb_ref, xt_ref, out_ref):
    xb = xb_ref[...]
    xt = xt_ref[...]
    dot = jnp.dot(xb, xt, preferred_element_type=jnp.float32)
    sqr = jnp.sum(xb * xb, axis=1, keepdims=True)
    sqc = jnp.sum(xt * xt, axis=0, keepdims=True)
    d = -2.0 * dot + sqr + sqc
    fio = lax.broadcasted_iota(jnp.int32, (KNN_R, N), 1).astype(jnp.float32)
    cols = []
    for _ in range(K):
        m = jnp.min(d, axis=1, keepdims=True)
        cand = jnp.min(jnp.where(d <= m, fio, jnp.float32(N)), axis=1,
                       keepdims=True)
        cols.append(cand)
        d = jnp.where(fio == cand, jnp.float32(jnp.inf), d)
    idx = jnp.concatenate(cols, axis=1).astype(jnp.int32)
    out_ref[...] = jnp.concatenate(
        [idx, jnp.zeros((KNN_R, 16 - K), jnp.int32)], axis=1)


def _knn(xyzT8_rows, x8T):
    nr = xyzT8_rows.shape[0]
    return pl.pallas_call(
        _knn_body,
        grid=(nr // KNN_R,),
        in_specs=[
            pl.BlockSpec((KNN_R, 8), lambda i: (i, 0)),
            pl.BlockSpec((8, N), lambda i: (0, 0)),
        ],
        out_specs=pl.BlockSpec((KNN_R, 16), lambda i: (i, 0)),
        out_shape=jax.ShapeDtypeStruct((nr, 16), jnp.int32),
        compiler_params=pltpu.CompilerParams(
            dimension_semantics=("parallel",)),
    )(xyzT8_rows, x8T)


def _sc_gather(table, idx_flat):
    b_total = idx_flat.shape[0]
    b_per_w = b_total // _NW
    n_ch = b_per_w // _GCH
    mesh = plsc.VectorSubcoreMesh(core_axis_name="c", subcore_axis_name="s")

    @functools.partial(
        pl.kernel,
        mesh=mesh,
        out_type=jax.ShapeDtypeStruct((b_total, D_TBL), jnp.float32),
        scratch_types=[
            pltpu.VMEM((_GCH,), jnp.int32),
            pltpu.VMEM((_GCH, D_TBL), jnp.float32),
            pltpu.SemaphoreType.DMA,
        ],
    )
    def gk(table_hbm, idx_hbm, out_hbm, idx_v, rows_v, sem):
        wid = lax.axis_index("s") * _NC + lax.axis_index("c")
        base = wid * b_per_w

        @pl.loop(0, n_ch)
        def _(c):
            off = base + c * _GCH
            pltpu.sync_copy(idx_hbm.at[pl.ds(off, _GCH)], idx_v)
            pltpu.async_copy(table_hbm.at[idx_v], rows_v, sem).wait()
            pltpu.sync_copy(rows_v, out_hbm.at[pl.ds(off, _GCH)])

    return gk(table, idx_flat)


def _mlp_body(g_ref, p1_ref, x_ref, uf_ref,
              wr_ref, wz_ref, wh_ref, wdr_ref, wdz_ref, wdh_ref,
              wr1_ref, wfro_ref, wfr_ref, wfz_ref, wz1_ref, wh1_ref, wfc_ref,
              br0_ref, br1_ref, bz0_ref, bz1_ref, bh0_ref, bh1_ref, bfc_ref,
              feats_ref, flow_ref):
    f32 = jnp.float32
    dot = lambda a, b: jnp.dot(a, b, preferred_element_type=f32)
    p1 = p1_ref[...]
    xb = x_ref[...]
    fr = dot(p1, wfr_ref[...])
    fz = dot(p1, wfz_ref[...])
    cr = dot(xb, wdr_ref[...])
    cz = dot(xb, wdz_ref[...])
    ch = dot(xb, wdh_ref[...])
    gall = g_ref[...].reshape(K * MLP_NB, D_TBL)
    aR = dot(gall, wr_ref[...]).reshape(K, MLP_NB, 64)
    aZ = dot(gall, wz_ref[...]).reshape(K, MLP_NB, 64)
    aH = dot(gall, wh_ref[...]).reshape(K, MLP_NB, 64)
    r = _leaky(aR + (fr - cr + br0_ref[...])[None])
    r = jax.nn.sigmoid(
        dot(r.reshape(K * MLP_NB, 64), wr1_ref[...]).reshape(K, MLP_NB, 64)
        + br1_ref[...][None])
    p1e = dot((r * p1[None]).reshape(K * MLP_NB, 64),
              wfro_ref[...]).reshape(K, MLP_NB, 64)
    h = _leaky(aH + (bh0_ref[...] - ch)[None] + p1e)
    hmax = jnp.max(h, axis=0)
    z = _leaky(aZ + (fz - cz + bz0_ref[...])[None])
    zmax = jnp.max(z, axis=0)
    z = jax.nn.sigmoid(dot(zmax, wz1_ref[...]) + bz1_ref[...])
    h = jnp.tanh(dot(hmax, wh1_ref[...]) + bh1_ref[...])
    feats = (1.0 - z) * p1 + z * h
    feats_ref[...] = feats
    fl = jnp.clip(dot(feats - p1, wfc_ref[...]) + bfc_ref[...], -200.0, 200.0)
    flow_ref[...] = fl + uf_ref[...]


def _mlp(g3, p1T, xyzT8, upfT8, weights):
    nr = p1T.shape[0]
    nblk = nr // MLP_NB
    full = lambda shape: pl.BlockSpec(shape, lambda i: tuple(0 for _ in shape))
    in_specs = [
        pl.BlockSpec((K, MLP_NB, D_TBL), lambda i: (0, i, 0)),
        pl.BlockSpec((MLP_NB, 64), lambda i: (i, 0)),
        pl.BlockSpec((MLP_NB, 8), lambda i: (i, 0)),
        pl.BlockSpec((MLP_NB, 8), lambda i: (i, 0)),
    ] + [full(w.shape) for w in weights]
    return pl.pallas_call(
        _mlp_body,
        grid=(nblk,),
        in_specs=in_specs,
        out_specs=[
            pl.BlockSpec((MLP_NB, 64), lambda i: (i, 0)),
            pl.BlockSpec((MLP_NB, 8), lambda i: (i, 0)),
        ],
        out_shape=[
            jax.ShapeDtypeStruct((nr, 64), jnp.float32),
            jax.ShapeDtypeStruct((nr, 8), jnp.float32),
        ],
        compiler_params=pltpu.CompilerParams(
            dimension_semantics=("parallel",)),
    )(g3, p1T, xyzT8, upfT8, *weights)


def kernel(pc1, pc2, feat1_new, feat2_new, feat1, feat2, up_flow, up_feat,
           W_r0, b_r0, W_r1, b_r1, W_z0, b_z0, W_z1, b_z1, W_h0, b_h0,
           W_h1, b_h1, Wfr, Wfro, Wfz, W_fc, b_fc):
    xyz = pc1[0].T
    xyzT8 = jnp.pad(xyz, ((0, 0), (0, 5)))
    x8T = xyzT8.T

    table = jnp.concatenate(
        [feat1[0].T, feat1_new[0].T, xyz,
         jnp.zeros((N, D_TBL - 131), jnp.float32)], axis=1)

    S = 4
    H = N // S
    idx16s = [_knn(xyzT8[s * H:(s + 1) * H], x8T) for s in range(S)]
    gs = [
        _sc_gather(table, ix[:, :K].T.reshape(-1)).reshape(K, H, D_TBL)
        for ix in idx16s
    ]

    p1T = up_feat[0].T
    upfT8 = jnp.pad(up_flow[0].T, ((0, 0), (0, 5)))

    pad13 = lambda w: jnp.pad(w.T, ((0, D_TBL - 131), (0, 0)))
    pad_d = lambda w: jnp.pad(w[:, 128:131].T, ((0, 5), (0, 0)))
    weights = [
        pad13(W_r0), pad13(W_z0), pad13(W_h0),
        pad_d(W_r0), pad_d(W_z0), pad_d(W_h0),
        W_r1.T, Wfro.T, Wfr.T, Wfz.T, W_z1.T, W_h1.T,
        jnp.pad(W_fc.T, ((0, 0), (0, 5))),
        b_r0.reshape(1, 64), b_r1.reshape(1, 64),
        b_z0.reshape(1, 64), b_z1.reshape(1, 64),
        b_h0.reshape(1, 64), b_h1.reshape(1, 64),
        jnp.pad(b_fc, (0, 5)).reshape(1, 8),
    ]
    outs = [
        _mlp(gs[s], p1T[s * H:(s + 1) * H], xyzT8[s * H:(s + 1) * H],
             upfT8[s * H:(s + 1) * H], weights)
        for s in range(S)
    ]

    feats = jnp.concatenate([o[0] for o in outs], axis=0)
    flow8 = jnp.concatenate([o[1] for o in outs], axis=0)
    feats_new = feats.T[None]
    flow = flow8[:, :3].T[None]
    return (feats_new, flow)


# speedup vs baseline: 1.0237x; 1.0237x over previous
import functools

import jax
import jax.numpy as jnp
from jax import lax
from jax.experimental import pallas as pl
from jax.experimental.pallas import tpu as pltpu
from jax.experimental.pallas import tpu_sc as plsc

N = 8192
K = 9
D_TBL = 256
KNN_R = 256
MLP_NB = 512

_NC, _NS = 2, 16
_NW = _NC * _NS
_GCH = 128


def _leaky(x):
    return jnp.where(x > 0, x, 0.1 * x)


def _knn_body(xb_ref, xt_ref, out_ref):
    xb = xb_ref[...]
    xt = xt_ref[...]
    dot = jnp.dot(xb, xt, preferred_element_type=jnp.float32)
    sqr = jnp.sum(xb * xb, axis=1, keepdims=True)
    sqc = jnp.sum(xt * xt, axis=0, keepdims=True)
    d = -2.0 * dot + sqr + sqc
    fio = lax.broadcasted_iota(jnp.int32, (KNN_R, N), 1).astype(jnp.float32)
    cols = []
    for _ in range(K):
        m = jnp.min(d, axis=1, keepdims=True)
        cand = jnp.min(jnp.where(d <= m, fio, jnp.float32(N)), axis=1,
                       keepdims=True)
        cols.append(cand)
        d = jnp.where(fio == cand, jnp.float32(jnp.inf), d)
    idx = jnp.concatenate(cols, axis=1).astype(jnp.int32)
    out_ref[...] = jnp.concatenate(
        [idx, jnp.zeros((KNN_R, 16 - K), jnp.int32)], axis=1)


def _knn(xyzT8_rows, x8T):
    nr = xyzT8_rows.shape[0]
    return pl.pallas_call(
        _knn_body,
        grid=(nr // KNN_R,),
        in_specs=[
            pl.BlockSpec((KNN_R, 8), lambda i: (i, 0)),
            pl.BlockSpec((8, N), lambda i: (0, 0)),
        ],
        out_specs=pl.BlockSpec((KNN_R, 16), lambda i: (i, 0)),
        out_shape=jax.ShapeDtypeStruct((nr, 16), jnp.int32),
        compiler_params=pltpu.CompilerParams(
            dimension_semantics=("parallel",)),
    )(xyzT8_rows, x8T)


def _sc_gather(table, idx_flat):
    b_total = idx_flat.shape[0]
    b_per_w = b_total // _NW
    n_ch = b_per_w // _GCH
    mesh = plsc.VectorSubcoreMesh(core_axis_name="c", subcore_axis_name="s")

    @functools.partial(
        pl.kernel,
        mesh=mesh,
        out_type=jax.ShapeDtypeStruct((b_total, D_TBL), jnp.float32),
        scratch_types=[
            pltpu.VMEM((_GCH,), jnp.int32),
            pltpu.VMEM((_GCH, D_TBL), jnp.float32),
            pltpu.SemaphoreType.DMA,
        ],
    )
    def gk(table_hbm, idx_hbm, out_hbm, idx_v, rows_v, sem):
        wid = lax.axis_index("s") * _NC + lax.axis_index("c")
        base = wid * b_per_w

        @pl.loop(0, n_ch)
        def _(c):
            off = base + c * _GCH
            pltpu.sync_copy(idx_hbm.at[pl.ds(off, _GCH)], idx_v)
            pltpu.async_copy(table_hbm.at[idx_v], rows_v, sem).wait()
            pltpu.sync_copy(rows_v, out_hbm.at[pl.ds(off, _GCH)])

    return gk(table, idx_flat)


def _mlp_body(g_ref, p1_ref, x_ref, uf_ref,
              wr_ref, wz_ref, wh_ref, wdr_ref, wdz_ref, wdh_ref,
              wr1_ref, wfro_ref, wfr_ref, wfz_ref, wz1_ref, wh1_ref, wfc_ref,
              br0_ref, br1_ref, bz0_ref, bz1_ref, bh0_ref, bh1_ref, bfc_ref,
              feats_ref, flow_ref):
    f32 = jnp.float32
    dot = lambda a, b: jnp.dot(a, b, preferred_element_type=f32)
    p1 = p1_ref[...]
    xb = x_ref[...]
    fr = dot(p1, wfr_ref[...])
    fz = dot(p1, wfz_ref[...])
    cr = dot(xb, wdr_ref[...])
    cz = dot(xb, wdz_ref[...])
    ch = dot(xb, wdh_ref[...])
    gall = g_ref[...].reshape(K * MLP_NB, D_TBL)
    aR = dot(gall, wr_ref[...]).reshape(K, MLP_NB, 64)
    aZ = dot(gall, wz_ref[...]).reshape(K, MLP_NB, 64)
    aH = dot(gall, wh_ref[...]).reshape(K, MLP_NB, 64)
    r = _leaky(aR + (fr - cr + br0_ref[...])[None])
    r = jax.nn.sigmoid(
        dot(r.reshape(K * MLP_NB, 64), wr1_ref[...]).reshape(K, MLP_NB, 64)
        + br1_ref[...][None])
    p1e = dot((r * p1[None]).reshape(K * MLP_NB, 64),
              wfro_ref[...]).reshape(K, MLP_NB, 64)
    h = _leaky(aH + (bh0_ref[...] - ch)[None] + p1e)
    hmax = jnp.max(h, axis=0)
    z = _leaky(aZ + (fz - cz + bz0_ref[...])[None])
    zmax = jnp.max(z, axis=0)
    z = jax.nn.sigmoid(dot(zmax, wz1_ref[...]) + bz1_ref[...])
    h = jnp.tanh(dot(hmax, wh1_ref[...]) + bh1_ref[...])
    feats = (1.0 - z) * p1 + z * h
    feats_ref[...] = feats
    fl = jnp.clip(dot(feats - p1, wfc_ref[...]) + bfc_ref[...], -200.0, 200.0)
    flow_ref[...] = fl + uf_ref[...]


def _mlp(g3, p1T, xyzT8, upfT8, weights):
    nr = p1T.shape[0]
    nblk = nr // MLP_NB
    full = lambda shape: pl.BlockSpec(shape, lambda i: tuple(0 for _ in shape))
    in_specs = [
        pl.BlockSpec((K, MLP_NB, D_TBL), lambda i: (0, i, 0)),
        pl.BlockSpec((MLP_NB, 64), lambda i: (i, 0)),
        pl.BlockSpec((MLP_NB, 8), lambda i: (i, 0)),
        pl.BlockSpec((MLP_NB, 8), lambda i: (i, 0)),
    ] + [full(w.shape) for w in weights]
    return pl.pallas_call(
        _mlp_body,
        grid=(nblk,),
        in_specs=in_specs,
        out_specs=[
            pl.BlockSpec((MLP_NB, 64), lambda i: (i, 0)),
            pl.BlockSpec((MLP_NB, 8), lambda i: (i, 0)),
        ],
        out_shape=[
            jax.ShapeDtypeStruct((nr, 64), jnp.float32),
            jax.ShapeDtypeStruct((nr, 8), jnp.float32),
        ],
        compiler_params=pltpu.CompilerParams(
            dimension_semantics=("parallel",)),
    )(g3, p1T, xyzT8, upfT8, *weights)


def kernel(pc1, pc2, feat1_new, feat2_new, feat1, feat2, up_flow, up_feat,
           W_r0, b_r0, W_r1, b_r1, W_z0, b_z0, W_z1, b_z1, W_h0, b_h0,
           W_h1, b_h1, Wfr, Wfro, Wfz, W_fc, b_fc):
    xyz = pc1[0].T
    xyzT8 = jnp.pad(xyz, ((0, 0), (0, 5)))
    x8T = xyzT8.T

    table = jnp.concatenate(
        [feat1[0].T, feat1_new[0].T, xyz,
         jnp.zeros((N, D_TBL - 131), jnp.float32)], axis=1)

    S = 2
    H = N // S
    idx16s = [_knn(xyzT8[s * H:(s + 1) * H], x8T) for s in range(S)]
    gs = [
        _sc_gather(table, ix[:, :K].T.reshape(-1)).reshape(K, H, D_TBL)
        for ix in idx16s
    ]

    p1T = up_feat[0].T
    upfT8 = jnp.pad(up_flow[0].T, ((0, 0), (0, 5)))

    pad13 = lambda w: jnp.pad(w.T, ((0, D_TBL - 131), (0, 0)))
    pad_d = lambda w: jnp.pad(w[:, 128:131].T, ((0, 5), (0, 0)))
    weights = [
        pad13(W_r0), pad13(W_z0), pad13(W_h0),
        pad_d(W_r0), pad_d(W_z0), pad_d(W_h0),
        W_r1.T, Wfro.T, Wfr.T, Wfz.T, W_z1.T, W_h1.T,
        jnp.pad(W_fc.T, ((0, 0), (0, 5))),
        b_r0.reshape(1, 64), b_r1.reshape(1, 64),
        b_z0.reshape(1, 64), b_z1.reshape(1, 64),
        b_h0.reshape(1, 64), b_h1.reshape(1, 64),
        jnp.pad(b_fc, (0, 5)).reshape(1, 8),
    ]
    outs = [
        _mlp(gs[s], p1T[s * H:(s + 1) * H], xyzT8[s * H:(s + 1) * H],
             upfT8[s * H:(s + 1) * H], weights)
        for s in range(S)
    ]

    feats = jnp.concatenate([o[0] for o in outs], axis=0)
    flow8 = jnp.concatenate([o[1] for o in outs], axis=0)
    feats_new = feats.T[None]
    flow = flow8[:, :3].T[None]
    return (feats_new, flow)
